# baseline (speedup 1.0000x reference)
import functools

import jax
import jax.numpy as jnp
from jax import lax
from jax.experimental import pallas as pl
from jax.experimental.pallas import tpu as pltpu
from jax.experimental.pallas import tpu_sc as plsc

N = 10000
F = 128
NC = 2
NS = 16
K = 128
HALF = N // NC
ACC_ROWS = HALF + NS
RPT = 312
TAIL = HALF - RPT * NS


def _spmm_sc(row, col, w, dense):
    e_pad = row.shape[0]
    per_tile = e_pad // NS
    nchunks = per_tile // K
    mesh = plsc.VectorSubcoreMesh(core_axis_name="c", subcore_axis_name="s")

    @functools.partial(
        pl.kernel,
        mesh=mesh,
        out_type=jax.ShapeDtypeStruct((N, F), jnp.float32),
        scratch_types=[
            pltpu.VMEM((K,), jnp.int32),
            pltpu.VMEM((K,), jnp.int32),
            pltpu.VMEM((K,), jnp.int32),
            pltpu.VMEM((K,), jnp.int32),
            pltpu.VMEM((K,), jnp.float32),
            pltpu.VMEM((K,), jnp.float32),
            pltpu.VMEM((K, F), jnp.float32),
            pltpu.VMEM((K, F), jnp.float32),
            pltpu.VMEM((RPT, F), jnp.float32),
            pltpu.VMEM_SHARED((ACC_ROWS, F), jnp.float32),
            pltpu.SemaphoreType.DMA,
            pltpu.SemaphoreType.DMA,
            pltpu.SemaphoreType.DMA,
            pltpu.SemaphoreType.DMA,
            pltpu.SemaphoreType.DMA,
            pltpu.SemaphoreType.DMA,
        ],
    )
    def spmm(row_hbm, col_hbm, w_hbm, dense_hbm, out_hbm,
             colv0, colv1, rowv0, rowv1, wv0, wv1, rows0, rows1, zbuf,
             acc, gsem0, gsem1, isem0, isem1, csem0, csem1):
        c = lax.axis_index("c")
        s = lax.axis_index("s")
        row_base = c * HALF
        colv = (colv0, colv1)
        rowv = (rowv0, rowv1)
        wv = (wv0, wv1)
        rows = (rows0, rows1)
        gsem = (gsem0, gsem1)
        isem = (isem0, isem1)
        csem = (csem0, csem1)

        def zero_body(i, _):
            r = i // (F // 16)
            j = i % (F // 16)
            zbuf[r, pl.ds(j * 16, 16)] = jnp.zeros((16,), jnp.float32)
            return 0
        lax.fori_loop(0, RPT * (F // 16), zero_body, 0)
        pltpu.sync_copy(zbuf, acc.at[pl.ds(s * RPT, RPT)])

        @pl.when(s == NS - 1)
        def _():
            pltpu.sync_copy(zbuf.at[pl.ds(0, ACC_ROWS - NS * RPT)],
                            acc.at[pl.ds(NS * RPT, ACC_ROWS - NS * RPT)])
        plsc.subcore_barrier()

        base = s * per_tile

        trash = HALF + s

        def scale(b):
            rowsv, rv_ref, wv_ref = rows[b], rowv[b], wv[b]

            def scale_body(eb, _):
                sl = pl.ds(eb * 16, 16)
                rv = rv_ref[sl] - row_base
                ok = (rv >= 0) & (rv < HALF)
                rv_ref[sl] = jnp.where(ok, rv, trash)
                wvec = wv_ref[sl]
                for i in range(16):
                    wb = jnp.full((16,), wvec[i], jnp.float32)
                    e = eb * 16 + i
                    for j in range(F // 16):
                        rowsv[e, pl.ds(j * 16, 16)] = (
                            rowsv[e, pl.ds(j * 16, 16)] * wb)
                return 0
            lax.fori_loop(0, K // 16, scale_body, 0)

        for b in range(2):
            off = base + b * K
            pltpu.sync_copy(col_hbm.at[pl.ds(off, K)], colv[b])
            pltpu.async_copy(dense_hbm.at[colv[b]], rows[b], gsem[b])
            pltpu.async_copy(row_hbm.at[pl.ds(off, K)], rowv[b], isem[b])
            pltpu.async_copy(w_hbm.at[pl.ds(off, K)], wv[b], isem[b])

        def pair_body(cc, _):
            for b in range(2):
                ci = cc * 2 + b
                nxt = ci + 2
                pltpu.make_async_copy(
                    dense_hbm.at[colv[b]], rows[b], gsem[b]).wait()

                @pl.when(nxt < nchunks)
                def _():
                    pltpu.async_copy(
                        col_hbm.at[pl.ds(base + nxt * K, K)],
                        colv[b], csem[b])

                pltpu.make_async_copy(
                    row_hbm.at[pl.ds(base, K)], rowv[b], isem[b]).wait()
                pltpu.make_async_copy(
                    w_hbm.at[pl.ds(base, K)], wv[b], isem[b]).wait()
                scale(b)
                pltpu.sync_copy(rows[b], acc.at[rowv[b]], add=True)

                @pl.when(nxt < nchunks)
                def _():
                    off = base + nxt * K
                    pltpu.make_async_copy(
                        col_hbm.at[pl.ds(base, K)], colv[b],
                        csem[b]).wait()
                    pltpu.async_copy(dense_hbm.at[colv[b]], rows[b],
                                     gsem[b])
                    pltpu.async_copy(row_hbm.at[pl.ds(off, K)], rowv[b],
                                     isem[b])
                    pltpu.async_copy(w_hbm.at[pl.ds(off, K)], wv[b],
                                     isem[b])
            return 0
        lax.fori_loop(0, nchunks // 2, pair_body, 0)

        plsc.subcore_barrier()
        pltpu.sync_copy(acc.at[pl.ds(s * RPT, RPT)],
                        out_hbm.at[pl.ds(row_base + s * RPT, RPT)])

        @pl.when(s == NS - 1)
        def _():
            pltpu.sync_copy(acc.at[pl.ds(NS * RPT, TAIL)],
                            out_hbm.at[pl.ds(row_base + NS * RPT, TAIL)])

    return spmm(row, col, w, dense)


def _mm_body(x_ref, w_ref, o_ref):
    o_ref[...] = jnp.dot(x_ref[...], w_ref[...],
                         preferred_element_type=jnp.float32)


def _mm(x, W):
    B = 1000
    return pl.pallas_call(
        _mm_body,
        grid=(N // B,),
        in_specs=[pl.BlockSpec((B, F), lambda i: (i, 0)),
                  pl.BlockSpec((F, F), lambda i: (0, 0))],
        out_specs=pl.BlockSpec((B, F), lambda i: (i, 0)),
        out_shape=jax.ShapeDtypeStruct((N, F), jnp.float32),
    )(x, W)


def _fuse_body(a_ref, w_ref, h_ref, s_ref):
    h = jnp.maximum(a_ref[...], 0.0)
    h_ref[...] = h
    s_ref[...] = jnp.dot(h, w_ref[...], preferred_element_type=jnp.float32)


def _fuse(a, W):
    B = 1000
    return pl.pallas_call(
        _fuse_body,
        grid=(N // B,),
        in_specs=[pl.BlockSpec((B, F), lambda i: (i, 0)),
                  pl.BlockSpec((F, F), lambda i: (0, 0))],
        out_specs=[pl.BlockSpec((B, F), lambda i: (i, 0)),
                   pl.BlockSpec((B, F), lambda i: (i, 0))],
        out_shape=[jax.ShapeDtypeStruct((N, F), jnp.float32),
                   jax.ShapeDtypeStruct((N, F), jnp.float32)],
    )(a, W)


def _final_body(a_ref, o_ref):
    z = jnp.maximum(a_ref[...], 0.0)
    m = jnp.max(z, axis=1, keepdims=True)
    ez = jnp.exp(z - m)
    lse = jnp.log(jnp.sum(ez, axis=1, keepdims=True))
    o_ref[...] = z - m - lse


def _final(a):
    B = 1000
    return pl.pallas_call(
        _final_body,
        grid=(N // B,),
        in_specs=[pl.BlockSpec((B, F), lambda i: (i, 0))],
        out_specs=pl.BlockSpec((B, F), lambda i: (i, 0)),
        out_shape=jax.ShapeDtypeStruct((N, F), jnp.float32),
    )(a)


def kernel(x, edge_index, edge_weight, W1, W2):
    row = edge_index[0]
    col = edge_index[1]
    e = row.shape[0]
    step = NS * K * 2
    e_pad = ((e + step - 1) // step) * step
    pad = e_pad - e
    if pad:
        row = jnp.concatenate([row, jnp.full((pad,), N, jnp.int32)])
        col = jnp.concatenate([col, jnp.zeros((pad,), jnp.int32)])
        edge_weight = jnp.concatenate(
            [edge_weight, jnp.zeros((pad,), jnp.float32)])

    support1 = _mm(x, W1)
    p1 = _spmm_sc(row, col, edge_weight, support1)
    h, support2 = _fuse(p1, W2)
    p2 = _spmm_sc(row, col, edge_weight, support2)
    out = _final(p2)
    return out, h
